# baseline (speedup 1.0000x reference)
import functools

import jax
import jax.numpy as jnp
from jax import lax
from jax.experimental import pallas as pl
from jax.experimental.pallas import tpu as pltpu
from jax.experimental.pallas import tpu_sc as plsc

_LS = 0.1
_SCORE_COLS = 64

_GATHER_DNUMS = lax.GatherDimensionNumbers(
    offset_dims=(), collapsed_slice_dims=(0,), start_index_map=(0,))


def _lane_shuffle(x, idx):
  return lax.gather(x, idx[:, None], _GATHER_DNUMS, slice_sizes=(1,),
                    mode=lax.GatherScatterMode.PROMISE_IN_BOUNDS)


def _allreduce16(x, lane):
  for sh in (8, 4, 2, 1):
    x = x + _lane_shuffle(x, lane ^ sh)
  return x


def _build_sc_scores(B, K, D, S, NBUF):
  mesh = plsc.VectorSubcoreMesh(core_axis_name="c", subcore_axis_name="s")
  NC, NS = mesh.num_cores, mesh.num_subcores
  NW = NC * NS
  per_tile = B // NW
  n_sub = per_tile // S
  n_vec = D // 16
  R = K + 1
  n_grp = (R + 15) // 16

  def body(center_hbm, context_hbm, neg_hbm, wc_hbm, wx_hbm, out_hbm,
           cidx, xidx, nidx, crows, xrows, rowbuf, scores_v, sem_r, sem_n):
    wid = lax.axis_index("s") * NC + lax.axis_index("c")
    tile_base = wid * per_tile
    lane = lax.iota(jnp.int32, 16)

    def sub_block(sb, carry):
      gbase = tile_base + sb * S
      for h in range(S // 128):
        pltpu.sync_copy(center_hbm.at[pl.ds(gbase + h * 128, 128)],
                        cidx.at[h])
        pltpu.sync_copy(context_hbm.at[pl.ds(gbase + h * 128, 128)],
                        xidx.at[h])
      pltpu.sync_copy(neg_hbm.at[pl.ds(gbase, S)], nidx)

      row_copies = []
      for h in range(S // 128):
        row_copies.append(pltpu.async_copy(
            wc_hbm.at[cidx.at[h]], crows.at[pl.ds(h * 128, 128)], sem_r))
        row_copies.append(pltpu.async_copy(
            wx_hbm.at[xidx.at[h]], xrows.at[pl.ds(h * 128, 128)], sem_r))
      for b in range(NBUF):
        pltpu.async_copy(wx_hbm.at[nidx.at[b]], rowbuf.at[b], sem_n.at[b])
      for cp in row_copies:
        cp.wait()

      def chunk(i, carry2):
        base = i * NBUF
        for b in range(NBUF):
          e = base + b
          pltpu.make_async_copy(wx_hbm.at[nidx.at[0]], rowbuf.at[b],
                                sem_n.at[b]).wait()
          c = [crows[e, pl.ds(16 * j, 16)] for j in range(n_vec)]
          for g in range(n_grp):
            svec = None
            for r in range(min(16, R - g * 16)):
              t = g * 16 + r
              if t == 0:
                opnd = lambda j: xrows[e, pl.ds(16 * j, 16)]
              else:
                opnd = lambda j, _k=t - 1: rowbuf[b, _k, pl.ds(16 * j, 16)]
              acc = c[0] * opnd(0)
              for j in range(1, n_vec):
                acc = acc + c[j] * opnd(j)
              s = _allreduce16(acc, lane)
              if svec is None:
                svec = s
              else:
                svec = jnp.where(lane == r, s, svec)
            scores_v[e, pl.ds(16 * g, 16)] = svec

          @pl.when(e + NBUF < S)
          def _refill():
            pltpu.async_copy(wx_hbm.at[nidx.at[e + NBUF]], rowbuf.at[b],
                             sem_n.at[b])
        return carry2

      lax.fori_loop(0, S // NBUF, chunk, 0)
      pltpu.sync_copy(scores_v, out_hbm.at[pl.ds(gbase, S)])
      return carry

    lax.fori_loop(0, n_sub, sub_block, 0)

  return pl.kernel(
      body,
      out_type=jax.ShapeDtypeStruct((B, _SCORE_COLS), jnp.float32),
      mesh=mesh,
      compiler_params=pltpu.CompilerParams(use_tc_tiling_on_sc=False,
                                           disable_bounds_checks=True),
      scratch_types=[
          pltpu.VMEM((S // 128, 128), jnp.int32),
          pltpu.VMEM((S // 128, 128), jnp.int32),
          pltpu.VMEM((S, K), jnp.int32),
          pltpu.VMEM((S, D), jnp.float32),
          pltpu.VMEM((S, D), jnp.float32),
          pltpu.VMEM((NBUF, K, D), jnp.float32),
          pltpu.VMEM((S, _SCORE_COLS), jnp.float32),
          pltpu.SemaphoreType.DMA,
          pltpu.SemaphoreType.DMA((NBUF,)),
      ],
  )


def _logsig(x):
  return jnp.minimum(x, 0.0) - jnp.log1p(jnp.exp(-jnp.abs(x)))


def _loss_body(K, sc_ref, out_ref):
  x = sc_ref[...]
  col = lax.broadcasted_iota(jnp.int32, x.shape, 1)
  neg_mask = (col >= 1) & (col <= K)
  xm = jnp.where(neg_mask, x, 0.0)
  t = _logsig(-xm) + _LS * xm
  neg_term = jnp.sum(jnp.where(neg_mask, t, 0.0), axis=1)
  p = x[:, 0]
  pos_term = _logsig(p) - _LS * p
  out_ref[...] = -(pos_term + neg_term)


def _build_tc_loss(B, K, C, BLK=2048):
  return pl.pallas_call(
      functools.partial(_loss_body, K),
      grid=(B // BLK,),
      in_specs=[pl.BlockSpec((BLK, C), lambda i: (i, 0))],
      out_specs=pl.BlockSpec((BLK,), lambda i: (i,)),
      out_shape=jax.ShapeDtypeStruct((B,), jnp.float32),
  )


def kernel(center, context, negatives, W_center, W_context):
  B = center.shape[0]
  K = negatives.shape[1]
  D = W_center.shape[1]
  H = B // 4
  sc_scores = _build_sc_scores(H, K, D, S=128, NBUF=4)
  tc_loss = _build_tc_loss(H, K, _SCORE_COLS)
  ci = center.astype(jnp.int32)
  xi = context.astype(jnp.int32)
  ni = negatives.astype(jnp.int32)
  losses = []
  for h in range(4):
    sl = slice(h * H, (h + 1) * H)
    scores = sc_scores(ci[sl], xi[sl], ni[sl], W_center, W_context)
    losses.append(tc_loss(scores))
  return jnp.concatenate(losses)


# speedup vs baseline: 1.0502x; 1.0502x over previous
import functools

import jax
import jax.numpy as jnp
from jax import lax
from jax.experimental import pallas as pl
from jax.experimental.pallas import tpu as pltpu
from jax.experimental.pallas import tpu_sc as plsc

_LS = 0.1
_SCORE_COLS = 64

_GATHER_DNUMS = lax.GatherDimensionNumbers(
    offset_dims=(), collapsed_slice_dims=(0,), start_index_map=(0,))


def _lane_shuffle(x, idx):
  return lax.gather(x, idx[:, None], _GATHER_DNUMS, slice_sizes=(1,),
                    mode=lax.GatherScatterMode.PROMISE_IN_BOUNDS)


def _allreduce16(x, lane):
  for sh in (8, 4, 2, 1):
    x = x + _lane_shuffle(x, lane ^ sh)
  return x


def _build_sc_scores(B, K, D, S, NBUF):
  mesh = plsc.VectorSubcoreMesh(core_axis_name="c", subcore_axis_name="s")
  NC, NS = mesh.num_cores, mesh.num_subcores
  NW = NC * NS
  per_tile = B // NW
  n_sub = per_tile // S
  n_vec = D // 16
  R = K + 1
  n_grp = (R + 15) // 16

  def body(center_hbm, context_hbm, neg_hbm, wc_hbm, wx_hbm, out_hbm,
           cidx, xidx, nidx, crows, xrows, rowbuf, scores_v, sem_r, sem_n):
    wid = lax.axis_index("s") * NC + lax.axis_index("c")
    tile_base = wid * per_tile
    lane = lax.iota(jnp.int32, 16)

    def sub_block(sb, carry):
      gbase = tile_base + sb * S
      for h in range(S // 128):
        pltpu.sync_copy(center_hbm.at[pl.ds(gbase + h * 128, 128)],
                        cidx.at[h])
        pltpu.sync_copy(context_hbm.at[pl.ds(gbase + h * 128, 128)],
                        xidx.at[h])
      pltpu.sync_copy(neg_hbm.at[pl.ds(gbase, S)], nidx)

      row_copies = []
      for h in range(S // 128):
        row_copies.append(pltpu.async_copy(
            wc_hbm.at[cidx.at[h]], crows.at[pl.ds(h * 128, 128)], sem_r))
        row_copies.append(pltpu.async_copy(
            wx_hbm.at[xidx.at[h]], xrows.at[pl.ds(h * 128, 128)], sem_r))
      for b in range(NBUF):
        pltpu.async_copy(wx_hbm.at[nidx.at[b]], rowbuf.at[b], sem_n.at[b])
      for cp in row_copies:
        cp.wait()

      def chunk(i, carry2):
        base = i * NBUF
        for b in range(NBUF):
          e = base + b
          pltpu.make_async_copy(wx_hbm.at[nidx.at[0]], rowbuf.at[b],
                                sem_n.at[b]).wait()
          c = [crows[e, pl.ds(16 * j, 16)] for j in range(n_vec)]
          for g in range(n_grp):
            svec = None
            for r in range(min(16, R - g * 16)):
              t = g * 16 + r
              if t == 0:
                opnd = lambda j: xrows[e, pl.ds(16 * j, 16)]
              else:
                opnd = lambda j, _k=t - 1: rowbuf[b, _k, pl.ds(16 * j, 16)]
              acc = c[0] * opnd(0)
              for j in range(1, n_vec):
                acc = acc + c[j] * opnd(j)
              s = _allreduce16(acc, lane)
              if svec is None:
                svec = s
              else:
                svec = jnp.where(lane == r, s, svec)
            scores_v[e, pl.ds(16 * g, 16)] = svec

          @pl.when(e + NBUF < S)
          def _refill():
            pltpu.async_copy(wx_hbm.at[nidx.at[e + NBUF]], rowbuf.at[b],
                             sem_n.at[b])
        return carry2

      lax.fori_loop(0, S // NBUF, chunk, 0)
      pltpu.sync_copy(scores_v, out_hbm.at[pl.ds(gbase, S)])
      return carry

    lax.fori_loop(0, n_sub, sub_block, 0)

  return pl.kernel(
      body,
      out_type=jax.ShapeDtypeStruct((B, _SCORE_COLS), jnp.float32),
      mesh=mesh,
      compiler_params=pltpu.CompilerParams(use_tc_tiling_on_sc=False,
                                           disable_bounds_checks=True),
      scratch_types=[
          pltpu.VMEM((S // 128, 128), jnp.int32),
          pltpu.VMEM((S // 128, 128), jnp.int32),
          pltpu.VMEM((S, K), jnp.int32),
          pltpu.VMEM((S, D), jnp.float32),
          pltpu.VMEM((S, D), jnp.float32),
          pltpu.VMEM((NBUF, K, D), jnp.float32),
          pltpu.VMEM((S, _SCORE_COLS), jnp.float32),
          pltpu.SemaphoreType.DMA,
          pltpu.SemaphoreType.DMA((NBUF,)),
      ],
  )


def _logsig(x):
  return jnp.minimum(x, 0.0) - jnp.log1p(jnp.exp(-jnp.abs(x)))


def _loss_body(K, sc_ref, out_ref):
  x = sc_ref[...]
  col = lax.broadcasted_iota(jnp.int32, x.shape, 1)
  neg_mask = (col >= 1) & (col <= K)
  xm = jnp.where(neg_mask, x, 0.0)
  t = _logsig(-xm) + _LS * xm
  neg_term = jnp.sum(jnp.where(neg_mask, t, 0.0), axis=1)
  p = x[:, 0]
  pos_term = _logsig(p) - _LS * p
  out_ref[...] = -(pos_term + neg_term)


def _build_tc_loss(B, K, C, BLK=2048):
  return pl.pallas_call(
      functools.partial(_loss_body, K),
      grid=(B // BLK,),
      in_specs=[pl.BlockSpec((BLK, C), lambda i: (i, 0))],
      out_specs=pl.BlockSpec((BLK,), lambda i: (i,)),
      out_shape=jax.ShapeDtypeStruct((B,), jnp.float32),
  )


def kernel(center, context, negatives, W_center, W_context):
  B = center.shape[0]
  K = negatives.shape[1]
  D = W_center.shape[1]
  H = B // 2
  sc_scores = _build_sc_scores(H, K, D, S=256, NBUF=4)
  tc_loss = _build_tc_loss(H, K, _SCORE_COLS)
  ci = center.astype(jnp.int32)
  xi = context.astype(jnp.int32)
  ni = negatives.astype(jnp.int32)
  losses = []
  for h in range(2):
    sl = slice(h * H, (h + 1) * H)
    scores = sc_scores(ci[sl], xi[sl], ni[sl], W_center, W_context)
    losses.append(tc_loss(scores))
  return jnp.concatenate(losses)


# speedup vs baseline: 1.0990x; 1.0465x over previous
import functools

import jax
import jax.numpy as jnp
from jax import lax
from jax.experimental import pallas as pl
from jax.experimental.pallas import tpu as pltpu
from jax.experimental.pallas import tpu_sc as plsc

_LS = 0.1
_SCORE_COLS = 64

_GATHER_DNUMS = lax.GatherDimensionNumbers(
    offset_dims=(), collapsed_slice_dims=(0,), start_index_map=(0,))


def _lane_shuffle(x, idx):
  return lax.gather(x, idx[:, None], _GATHER_DNUMS, slice_sizes=(1,),
                    mode=lax.GatherScatterMode.PROMISE_IN_BOUNDS)


def _allreduce16(x, lane):
  for sh in (8, 4, 2, 1):
    x = x + _lane_shuffle(x, lane ^ sh)
  return x


def _build_sc_scores(B, K, D, S, NBUF):
  mesh = plsc.VectorSubcoreMesh(core_axis_name="c", subcore_axis_name="s")
  NC, NS = mesh.num_cores, mesh.num_subcores
  NW = NC * NS
  per_tile = B // NW
  n_sub = per_tile // S
  n_vec = D // 16
  R = K + 1
  n_grp = (R + 15) // 16

  def body(center_hbm, context_hbm, neg_hbm, wc_hbm, wx_hbm, out_hbm,
           cidx, xidx, nidx, crows, xrows, rowbuf, scores_v, sem_r, sem_n):
    wid = lax.axis_index("s") * NC + lax.axis_index("c")
    tile_base = wid * per_tile
    lane = lax.iota(jnp.int32, 16)

    def sub_block(sb, carry):
      gbase = tile_base + sb * S
      for h in range(S // 128):
        pltpu.sync_copy(center_hbm.at[pl.ds(gbase + h * 128, 128)],
                        cidx.at[h])
        pltpu.sync_copy(context_hbm.at[pl.ds(gbase + h * 128, 128)],
                        xidx.at[h])
      pltpu.sync_copy(neg_hbm.at[pl.ds(gbase, S)], nidx)

      row_copies = []
      for h in range(S // 128):
        row_copies.append(pltpu.async_copy(
            wc_hbm.at[cidx.at[h]], crows.at[pl.ds(h * 128, 128)], sem_r))
        row_copies.append(pltpu.async_copy(
            wx_hbm.at[xidx.at[h]], xrows.at[pl.ds(h * 128, 128)], sem_r))
      for b in range(NBUF):
        pltpu.async_copy(wx_hbm.at[nidx.at[b]], rowbuf.at[b], sem_n.at[b])
      for cp in row_copies:
        cp.wait()

      def chunk(i, carry2):
        base = i * NBUF
        for b in range(NBUF):
          e = base + b
          pltpu.make_async_copy(wx_hbm.at[nidx.at[0]], rowbuf.at[b],
                                sem_n.at[b]).wait()
          c = [crows[e, pl.ds(16 * j, 16)] for j in range(n_vec)]

          def dot_partial(t):
            if t == 0:
              opnd = lambda j: xrows[e, pl.ds(16 * j, 16)]
            else:
              opnd = lambda j, _k=t - 1: rowbuf[b, _k, pl.ds(16 * j, 16)]
            acc = c[0] * opnd(0)
            for j in range(1, n_vec):
              acc = acc + c[j] * opnd(j)
            return acc

          for g in range(n_grp):
            n_r = min(16, R - g * 16)
            svec = None
            if n_r == 16:
              for r in range(8):
                aa = dot_partial(g * 16 + r)
                ab = dot_partial(g * 16 + r + 8)
                z = jnp.where(lane < 8,
                              aa + _lane_shuffle(aa, lane ^ 8),
                              ab + _lane_shuffle(ab, lane ^ 8))
                for sh in (4, 2, 1):
                  z = z + _lane_shuffle(z, lane ^ sh)
                if svec is None:
                  svec = z
                else:
                  svec = jnp.where((lane == r) | (lane == r + 8), z, svec)
            else:
              for r in range(n_r):
                s = _allreduce16(dot_partial(g * 16 + r), lane)
                if svec is None:
                  svec = s
                else:
                  svec = jnp.where(lane == r, s, svec)
            scores_v[e, pl.ds(16 * g, 16)] = svec

          @pl.when(e + NBUF < S)
          def _refill():
            pltpu.async_copy(wx_hbm.at[nidx.at[e + NBUF]], rowbuf.at[b],
                             sem_n.at[b])
        return carry2

      lax.fori_loop(0, S // NBUF, chunk, 0)
      pltpu.sync_copy(scores_v, out_hbm.at[pl.ds(gbase, S)])
      return carry

    lax.fori_loop(0, n_sub, sub_block, 0)

  return pl.kernel(
      body,
      out_type=jax.ShapeDtypeStruct((B, _SCORE_COLS), jnp.float32),
      mesh=mesh,
      compiler_params=pltpu.CompilerParams(use_tc_tiling_on_sc=False,
                                           disable_bounds_checks=True),
      scratch_types=[
          pltpu.VMEM((S // 128, 128), jnp.int32),
          pltpu.VMEM((S // 128, 128), jnp.int32),
          pltpu.VMEM((S, K), jnp.int32),
          pltpu.VMEM((S, D), jnp.float32),
          pltpu.VMEM((S, D), jnp.float32),
          pltpu.VMEM((NBUF, K, D), jnp.float32),
          pltpu.VMEM((S, _SCORE_COLS), jnp.float32),
          pltpu.SemaphoreType.DMA,
          pltpu.SemaphoreType.DMA((NBUF,)),
      ],
  )


def _logsig(x):
  return jnp.minimum(x, 0.0) - jnp.log1p(jnp.exp(-jnp.abs(x)))


def _loss_body(K, sc_ref, out_ref):
  x = sc_ref[...]
  col = lax.broadcasted_iota(jnp.int32, x.shape, 1)
  neg_mask = (col >= 1) & (col <= K)
  xm = jnp.where(neg_mask, x, 0.0)
  t = _logsig(-xm) + _LS * xm
  neg_term = jnp.sum(jnp.where(neg_mask, t, 0.0), axis=1)
  p = x[:, 0]
  pos_term = _logsig(p) - _LS * p
  out_ref[...] = -(pos_term + neg_term)


def _build_tc_loss(B, K, C, BLK=2048):
  return pl.pallas_call(
      functools.partial(_loss_body, K),
      grid=(B // BLK,),
      in_specs=[pl.BlockSpec((BLK, C), lambda i: (i, 0))],
      out_specs=pl.BlockSpec((BLK,), lambda i: (i,)),
      out_shape=jax.ShapeDtypeStruct((B,), jnp.float32),
  )


def kernel(center, context, negatives, W_center, W_context):
  B = center.shape[0]
  K = negatives.shape[1]
  D = W_center.shape[1]
  H = B // 2
  sc_scores = _build_sc_scores(H, K, D, S=256, NBUF=4)
  tc_loss = _build_tc_loss(H, K, _SCORE_COLS)
  ci = center.astype(jnp.int32)
  xi = context.astype(jnp.int32)
  ni = negatives.astype(jnp.int32)
  losses = []
  for h in range(2):
    sl = slice(h * H, (h + 1) * H)
    scores = sc_scores(ci[sl], xi[sl], ni[sl], W_center, W_context)
    losses.append(tc_loss(scores))
  return jnp.concatenate(losses)
